# baseline (speedup 1.0000x reference)
import functools

import jax
import jax.numpy as jnp
from jax import lax
from jax.experimental import pallas as pl
from jax.experimental.pallas import tpu as pltpu
from jax.experimental.pallas import tpu_sc as plsc

_NC = 2
_NS = 16
_NW = _NC * _NS
_K = 80
_NBUF = 4
_PF = 2
_IPF = 3


def _segment_sum_sc(table, src, dst, n_out):
    e = src.shape[0]
    d = table.shape[1]
    per_w = e // _NW
    n_chunks = per_w // _K
    assert per_w * _NW == e and n_chunks * _K == per_w
    rows_per_tile = (n_out // _NS) // 8 * 8
    tail_rows = n_out - rows_per_tile * _NS
    assert 0 <= tail_rows <= rows_per_tile or rows_per_tile == 0

    mesh = plsc.VectorSubcoreMesh(core_axis_name="c", subcore_axis_name="s")

    @functools.partial(
        pl.kernel,
        mesh=mesh,
        out_type=jax.ShapeDtypeStruct((_NC, n_out, d), jnp.float32),
        scratch_types=[
            pltpu.VMEM_SHARED((n_out, d), jnp.float32),
            [pltpu.VMEM((_K,), jnp.int32) for _ in range(_NBUF)],
            [pltpu.VMEM((_K,), jnp.int32) for _ in range(_NBUF)],
            [pltpu.VMEM((_K, d), jnp.float32) for _ in range(_NBUF)],
            [pltpu.SemaphoreType.DMA for _ in range(_NBUF)],
            [pltpu.SemaphoreType.DMA for _ in range(_NBUF)],
            [pltpu.SemaphoreType.DMA for _ in range(_NBUF)],
            [pltpu.SemaphoreType.DMA for _ in range(_NBUF)],
        ],
    )
    def seg_kernel(table_h, src_h, dst_h, zero_h, out_h, acc, sidx,
                   didx, rows, isem, gsem, dsem, ssem):
        cid = lax.axis_index("c")
        sid = lax.axis_index("s")
        wid = sid * _NC + cid
        ebase = wid * per_w

        def start_sidx(c, b):
            pltpu.async_copy(src_h.at[pl.ds(ebase + c * _K, _K)], sidx[b],
                             isem[b])

        def wait_sidx(c, b):
            pltpu.make_async_copy(src_h.at[pl.ds(ebase + c * _K, _K)],
                                  sidx[b], isem[b]).wait()

        def start_fetch(c, b):
            pltpu.async_copy(table_h.at[sidx[b]], rows[b], gsem[b])
            pltpu.async_copy(dst_h.at[pl.ds(ebase + c * _K, _K)], didx[b],
                             dsem[b])

        def wait_fetch(c, b):
            pltpu.make_async_copy(table_h.at[sidx[b]], rows[b], gsem[b]).wait()
            pltpu.make_async_copy(dst_h.at[pl.ds(ebase + c * _K, _K)],
                                  didx[b], dsem[b]).wait()

        def wait_scatter(b):
            pltpu.make_async_copy(rows[b], acc.at[didx[b]], ssem[b]).wait()

        for c in range(_IPF):
            start_sidx(c, c)
        for c in range(_PF):
            wait_sidx(c, c)
            start_fetch(c, c)
        pltpu.sync_copy(zero_h,
                        acc.at[pl.ds(sid * rows_per_tile, rows_per_tile)])
        if tail_rows:
            @pl.when(sid == _NS - 1)
            def _():
                pltpu.sync_copy(
                    zero_h.at[pl.ds(0, tail_rows)],
                    acc.at[pl.ds(_NS * rows_per_tile, tail_rows)])
        plsc.subcore_barrier()

        n_main = (n_chunks - _PF) // _NBUF

        def body(i, carry):
            for b in range(_NBUF):
                c = _NBUF * i + b
                wait_fetch(c, b)
                pltpu.async_copy(rows[b], acc.at[didx[b]], ssem[b], add=True)
                bi = (b + _IPF) % _NBUF
                start_sidx(c + _IPF, bi)
                bp = (b + _PF) % _NBUF
                if b + _PF < _NBUF:
                    @pl.when(i > 0)
                    def _():
                        wait_scatter(bp)
                else:
                    wait_scatter(bp)
                wait_sidx(c + _PF, bp)
                start_fetch(c + _PF, bp)
            return carry

        lax.fori_loop(0, n_main, body, 0)
        for c in range(_NBUF * n_main, n_chunks):
            b = c % _NBUF
            wait_fetch(c, b)
            pltpu.async_copy(rows[b], acc.at[didx[b]], ssem[b], add=True)
            if c + _IPF < n_chunks:
                start_sidx(c + _IPF, (c + _IPF) % _NBUF)
            if c + _PF < n_chunks:
                bp = (c + _PF) % _NBUF
                wait_scatter(bp)
                wait_sidx(c + _PF, bp)
                start_fetch(c + _PF, bp)
        for b in range(_NBUF):
            wait_scatter(b)
        plsc.subcore_barrier()
        pltpu.sync_copy(acc.at[pl.ds(sid * rows_per_tile, rows_per_tile)],
                        out_h.at[cid, pl.ds(sid * rows_per_tile,
                                            rows_per_tile)])
        if tail_rows:
            @pl.when(sid == _NS - 1)
            def _():
                pltpu.sync_copy(acc.at[pl.ds(_NS * rows_per_tile, tail_rows)],
                                out_h.at[cid, pl.ds(_NS * rows_per_tile,
                                                    tail_rows)])

    zeros = jnp.zeros((max(rows_per_tile, tail_rows), d), jnp.float32)
    return seg_kernel(table, src, dst, zeros)


def _gin_mlp(x, partials, w1, b1, g, bb, w2, b2, eps):
    n, d = x.shape

    def body(x_ref, p_ref, w1_ref, b1_ref, g_ref, bb_ref, w2_ref, b2_ref,
             eps_ref, out_ref):
        xin = (1.0 + eps_ref[0]) * x_ref[...] + p_ref[0] + p_ref[1]
        h = lax.dot_general(xin, w1_ref[...], (((1,), (1,)), ((), ())),
                            preferred_element_type=jnp.float32) + b1_ref[...]
        mu = jnp.mean(h, axis=0, keepdims=True)
        var = jnp.mean((h - mu) ** 2, axis=0, keepdims=True)
        hn = (h - mu) * lax.rsqrt(var + 1e-5) * g_ref[...] + bb_ref[...]
        hr = jnp.maximum(hn, 0.0)
        out_ref[...] = lax.dot_general(hr, w2_ref[...], (((1,), (1,)), ((), ())),
                                       preferred_element_type=jnp.float32
                                       ) + b2_ref[...]

    vspec = pl.BlockSpec(memory_space=pltpu.MemorySpace.VMEM)
    sspec = pl.BlockSpec(memory_space=pltpu.MemorySpace.SMEM)
    return pl.pallas_call(
        body,
        out_shape=jax.ShapeDtypeStruct((n, d), jnp.float32),
        in_specs=[vspec] * 8 + [sspec],
        out_specs=vspec,
    )(x, partials, w1, b1, g, bb, w2, b2, eps)


def _proj_gin_mlp(x, wp, bp, partials, w1, b1, g, bb, w2, b2, eps):
    n, d = x.shape

    def body(x_ref, wp_ref, bp_ref, p_ref, w1_ref, b1_ref, g_ref, bb_ref,
             w2_ref, b2_ref, eps_ref, out_ref):
        xp = lax.dot_general(x_ref[...], wp_ref[...], (((1,), (1,)), ((), ())),
                             preferred_element_type=jnp.float32) + bp_ref[...]
        xin = (1.0 + eps_ref[0]) * xp + p_ref[0] + p_ref[1]
        h = lax.dot_general(xin, w1_ref[...], (((1,), (1,)), ((), ())),
                            preferred_element_type=jnp.float32) + b1_ref[...]
        mu = jnp.mean(h, axis=0, keepdims=True)
        var = jnp.mean((h - mu) ** 2, axis=0, keepdims=True)
        hn = (h - mu) * lax.rsqrt(var + 1e-5) * g_ref[...] + bb_ref[...]
        hr = jnp.maximum(hn, 0.0)
        out_ref[...] = lax.dot_general(hr, w2_ref[...], (((1,), (1,)), ((), ())),
                                       preferred_element_type=jnp.float32
                                       ) + b2_ref[...]

    vspec = pl.BlockSpec(memory_space=pltpu.MemorySpace.VMEM)
    sspec = pl.BlockSpec(memory_space=pltpu.MemorySpace.SMEM)
    return pl.pallas_call(
        body,
        out_shape=jax.ShapeDtypeStruct((n, d), jnp.float32),
        in_specs=[vspec] * 10 + [sspec],
        out_specs=vspec,
    )(x, wp, bp, partials, w1, b1, g, bb, w2, b2, eps)


def kernel(feat_user, feat_item, edge_ui, edge_iu, W_proj, b_proj,
           ui_W1, ui_b1, ui_g, ui_bb, ui_W2, ui_b2,
           ret_W1, ret_b1, ret_g, ret_bb, ret_W2, ret_b2,
           eps_ui, eps_ret):
    n = feat_user.shape[0]
    p_item = _segment_sum_sc(feat_user, edge_ui[0], edge_ui[1], n)
    h_item = _gin_mlp(feat_item, p_item, ui_W1, ui_b1, ui_g, ui_bb,
                      ui_W2, ui_b2, eps_ui)
    p_user = _segment_sum_sc(h_item, edge_iu[0], edge_iu[1], n)
    h_user = _proj_gin_mlp(feat_user, W_proj, b_proj, p_user, ret_W1, ret_b1,
                           ret_g, ret_bb, ret_W2, ret_b2, eps_ret)
    return (h_user, h_item)


# speedup vs baseline: 1.0329x; 1.0329x over previous
import functools

import jax
import jax.numpy as jnp
from jax import lax
from jax.experimental import pallas as pl
from jax.experimental.pallas import tpu as pltpu
from jax.experimental.pallas import tpu_sc as plsc

_NC = 2
_NS = 16
_NW = _NC * _NS
_K = 80
_NBUF = 3
_PF = 2
_ZR = 48


def _segment_sum_sc(table, src, dst, n_out):
    e = src.shape[0]
    d = table.shape[1]
    per_w = e // _NW
    n_chunks = per_w // _K
    assert per_w * _NW == e and n_chunks * _K == per_w
    rows_per_tile = (n_out // _NS) // 8 * 8
    tail_rows = n_out - rows_per_tile * _NS
    assert rows_per_tile % _ZR == 0 and tail_rows % 8 == 0

    mesh = plsc.VectorSubcoreMesh(core_axis_name="c", subcore_axis_name="s")

    @functools.partial(
        pl.kernel,
        mesh=mesh,
        out_type=jax.ShapeDtypeStruct((_NC, n_out, d), jnp.float32),
        scratch_types=[
            pltpu.VMEM_SHARED((n_out, d), jnp.float32),
            pltpu.VMEM((per_w,), jnp.int32),
            pltpu.VMEM((_ZR, d), jnp.float32),
            [pltpu.VMEM((_K,), jnp.int32) for _ in range(_NBUF)],
            [pltpu.VMEM((_K, d), jnp.float32) for _ in range(_NBUF)],
            [pltpu.SemaphoreType.DMA for _ in range(_NBUF)],
            [pltpu.SemaphoreType.DMA for _ in range(_NBUF)],
            [pltpu.SemaphoreType.DMA for _ in range(_NBUF)],
        ],
    )
    def seg_kernel(table_h, src_h, dst_h, out_h, acc, sidx, zblk,
                   didx, rows, gsem, dsem, ssem):
        cid = lax.axis_index("c")
        sid = lax.axis_index("s")
        wid = sid * _NC + cid
        ebase = wid * per_w
        icp0 = pltpu.async_copy(src_h.at[pl.ds(ebase, per_w)], sidx, gsem[0])
        z16 = jnp.zeros((16,), jnp.float32)

        def zrow(i, carry):
            for j in range(d // 16):
                zblk[i, pl.ds(j * 16, 16)] = z16
            return carry

        lax.fori_loop(0, _ZR, zrow, 0)
        for k in range(rows_per_tile // _ZR):
            pltpu.sync_copy(zblk,
                            acc.at[pl.ds(sid * rows_per_tile + k * _ZR, _ZR)])
        if tail_rows:
            @pl.when(sid == _NS - 1)
            def _():
                pltpu.sync_copy(
                    zblk.at[pl.ds(0, tail_rows)],
                    acc.at[pl.ds(_NS * rows_per_tile, tail_rows)])
        icp0.wait()

        def start_fetch(c, b):
            pltpu.async_copy(table_h.at[sidx.at[pl.ds(c * _K, _K)]], rows[b],
                             gsem[b])
            pltpu.async_copy(dst_h.at[pl.ds(ebase + c * _K, _K)], didx[b],
                             dsem[b])

        def wait_fetch(c, b):
            pltpu.make_async_copy(table_h.at[sidx.at[pl.ds(c * _K, _K)]],
                                  rows[b], gsem[b]).wait()
            pltpu.make_async_copy(dst_h.at[pl.ds(ebase + c * _K, _K)],
                                  didx[b], dsem[b]).wait()

        def wait_scatter(b):
            pltpu.make_async_copy(rows[b], acc.at[didx[b]], ssem[b]).wait()

        for c in range(_PF):
            start_fetch(c, c)
        plsc.subcore_barrier()
        n_main = (n_chunks - _PF) // _NBUF

        def body(i, carry):
            for b in range(_NBUF):
                c = _NBUF * i + b
                wait_fetch(c, b)
                pltpu.async_copy(rows[b], acc.at[didx[b]], ssem[b], add=True)
                bp = (b + _PF) % _NBUF
                if b + _PF < _NBUF:
                    @pl.when(i > 0)
                    def _():
                        wait_scatter(bp)
                else:
                    wait_scatter(bp)
                start_fetch(c + _PF, bp)
            return carry

        lax.fori_loop(0, n_main, body, 0)
        for c in range(_NBUF * n_main, n_chunks):
            b = c % _NBUF
            wait_fetch(c, b)
            pltpu.async_copy(rows[b], acc.at[didx[b]], ssem[b], add=True)
            if c + _PF < n_chunks:
                bp = (c + _PF) % _NBUF
                wait_scatter(bp)
                start_fetch(c + _PF, bp)
        for b in range(_NBUF):
            wait_scatter(b)
        plsc.subcore_barrier()
        pltpu.sync_copy(acc.at[pl.ds(sid * rows_per_tile, rows_per_tile)],
                        out_h.at[cid, pl.ds(sid * rows_per_tile,
                                            rows_per_tile)])
        if tail_rows:
            @pl.when(sid == _NS - 1)
            def _():
                pltpu.sync_copy(acc.at[pl.ds(_NS * rows_per_tile, tail_rows)],
                                out_h.at[cid, pl.ds(_NS * rows_per_tile,
                                                    tail_rows)])

    return seg_kernel(table, src, dst)


def _gin_mlp(x, partials, w1, b1, g, bb, w2, b2, eps):
    n, d = x.shape

    def body(x_ref, p_ref, w1_ref, b1_ref, g_ref, bb_ref, w2_ref, b2_ref,
             eps_ref, out_ref):
        xin = (1.0 + eps_ref[0]) * x_ref[...] + p_ref[0] + p_ref[1]
        h = lax.dot_general(xin, w1_ref[...], (((1,), (1,)), ((), ())),
                            preferred_element_type=jnp.float32) + b1_ref[...]
        mu = jnp.mean(h, axis=0, keepdims=True)
        var = jnp.mean((h - mu) ** 2, axis=0, keepdims=True)
        hn = (h - mu) * lax.rsqrt(var + 1e-5) * g_ref[...] + bb_ref[...]
        hr = jnp.maximum(hn, 0.0)
        out_ref[...] = lax.dot_general(hr, w2_ref[...], (((1,), (1,)), ((), ())),
                                       preferred_element_type=jnp.float32
                                       ) + b2_ref[...]

    vspec = pl.BlockSpec(memory_space=pltpu.MemorySpace.VMEM)
    sspec = pl.BlockSpec(memory_space=pltpu.MemorySpace.SMEM)
    return pl.pallas_call(
        body,
        out_shape=jax.ShapeDtypeStruct((n, d), jnp.float32),
        in_specs=[vspec] * 8 + [sspec],
        out_specs=vspec,
    )(x, partials, w1, b1, g, bb, w2, b2, eps)


def _proj_gin_mlp(x, wp, bp, partials, w1, b1, g, bb, w2, b2, eps):
    n, d = x.shape

    def body(x_ref, wp_ref, bp_ref, p_ref, w1_ref, b1_ref, g_ref, bb_ref,
             w2_ref, b2_ref, eps_ref, out_ref):
        xp = lax.dot_general(x_ref[...], wp_ref[...], (((1,), (1,)), ((), ())),
                             preferred_element_type=jnp.float32) + bp_ref[...]
        xin = (1.0 + eps_ref[0]) * xp + p_ref[0] + p_ref[1]
        h = lax.dot_general(xin, w1_ref[...], (((1,), (1,)), ((), ())),
                            preferred_element_type=jnp.float32) + b1_ref[...]
        mu = jnp.mean(h, axis=0, keepdims=True)
        var = jnp.mean((h - mu) ** 2, axis=0, keepdims=True)
        hn = (h - mu) * lax.rsqrt(var + 1e-5) * g_ref[...] + bb_ref[...]
        hr = jnp.maximum(hn, 0.0)
        out_ref[...] = lax.dot_general(hr, w2_ref[...], (((1,), (1,)), ((), ())),
                                       preferred_element_type=jnp.float32
                                       ) + b2_ref[...]

    vspec = pl.BlockSpec(memory_space=pltpu.MemorySpace.VMEM)
    sspec = pl.BlockSpec(memory_space=pltpu.MemorySpace.SMEM)
    return pl.pallas_call(
        body,
        out_shape=jax.ShapeDtypeStruct((n, d), jnp.float32),
        in_specs=[vspec] * 10 + [sspec],
        out_specs=vspec,
    )(x, wp, bp, partials, w1, b1, g, bb, w2, b2, eps)


def kernel(feat_user, feat_item, edge_ui, edge_iu, W_proj, b_proj,
           ui_W1, ui_b1, ui_g, ui_bb, ui_W2, ui_b2,
           ret_W1, ret_b1, ret_g, ret_bb, ret_W2, ret_b2,
           eps_ui, eps_ret):
    n = feat_user.shape[0]
    p_item = _segment_sum_sc(feat_user, edge_ui[0], edge_ui[1], n)
    h_item = _gin_mlp(feat_item, p_item, ui_W1, ui_b1, ui_g, ui_bb,
                      ui_W2, ui_b2, eps_ui)
    p_user = _segment_sum_sc(h_item, edge_iu[0], edge_iu[1], n)
    h_user = _proj_gin_mlp(feat_user, W_proj, b_proj, p_user, ret_W1, ret_b1,
                           ret_g, ret_bb, ret_W2, ret_b2, eps_ret)
    return (h_user, h_item)


# speedup vs baseline: 1.0402x; 1.0071x over previous
import functools

import jax
import jax.numpy as jnp
from jax import lax
from jax.experimental import pallas as pl
from jax.experimental.pallas import tpu as pltpu
from jax.experimental.pallas import tpu_sc as plsc

_NC = 2
_NS = 16
_NW = _NC * _NS
_K = 96
_NBUF = 3
_PF = 2
_ZR = 48


def _segment_sum_sc(table, src, dst, n_out):
    e = src.shape[0]
    d = table.shape[1]
    per_w = e // _NW
    n_full = per_w // _K
    tail_e = per_w - n_full * _K
    assert per_w * _NW == e and tail_e % 8 == 0 and 0 < tail_e < _K
    rows_per_tile = (n_out // _NS) // 8 * 8
    tail_rows = n_out - rows_per_tile * _NS
    assert rows_per_tile % _ZR == 0 and tail_rows % 8 == 0

    mesh = plsc.VectorSubcoreMesh(core_axis_name="c", subcore_axis_name="s")

    @functools.partial(
        pl.kernel,
        mesh=mesh,
        out_type=jax.ShapeDtypeStruct((_NC, n_out, d), jnp.float32),
        scratch_types=[
            pltpu.VMEM_SHARED((n_out, d), jnp.float32),
            pltpu.VMEM((per_w,), jnp.int32),
            [pltpu.VMEM((_K,), jnp.int32) for _ in range(_NBUF)],
            [pltpu.VMEM((_K, d), jnp.float32) for _ in range(_NBUF)],
            pltpu.VMEM((tail_e,), jnp.int32),
            pltpu.VMEM((tail_e, d), jnp.float32),
            [pltpu.SemaphoreType.DMA for _ in range(_NBUF)],
            [pltpu.SemaphoreType.DMA for _ in range(_NBUF)],
            [pltpu.SemaphoreType.DMA for _ in range(_NBUF)],
            [pltpu.SemaphoreType.DMA for _ in range(3)],
        ],
    )
    def seg_kernel(table_h, src_h, dst_h, out_h, acc, sidx,
                   didx, rows, didx_t, rows_t, gsem, dsem, ssem, tsem):
        cid = lax.axis_index("c")
        sid = lax.axis_index("s")
        wid = sid * _NC + cid
        ebase = wid * per_w
        icp0 = pltpu.async_copy(src_h.at[pl.ds(ebase, per_w)], sidx, gsem[0])
        z16 = jnp.zeros((16,), jnp.float32)

        def zrow(i, carry):
            for j in range(d // 16):
                rows[0][i, pl.ds(j * 16, 16)] = z16
            return carry

        lax.fori_loop(0, _ZR, zrow, 0)
        for k in range(rows_per_tile // _ZR):
            pltpu.sync_copy(rows[0].at[pl.ds(0, _ZR)],
                            acc.at[pl.ds(sid * rows_per_tile + k * _ZR, _ZR)])
        if tail_rows:
            @pl.when(sid == _NS - 1)
            def _():
                pltpu.sync_copy(
                    rows[0].at[pl.ds(0, tail_rows)],
                    acc.at[pl.ds(_NS * rows_per_tile, tail_rows)])
        icp0.wait()

        def start_fetch(c, b):
            pltpu.async_copy(table_h.at[sidx.at[pl.ds(c * _K, _K)]], rows[b],
                             gsem[b])
            pltpu.async_copy(dst_h.at[pl.ds(ebase + c * _K, _K)], didx[b],
                             dsem[b])

        def wait_fetch(c, b):
            pltpu.make_async_copy(table_h.at[sidx.at[pl.ds(c * _K, _K)]],
                                  rows[b], gsem[b]).wait()
            pltpu.make_async_copy(dst_h.at[pl.ds(ebase + c * _K, _K)],
                                  didx[b], dsem[b]).wait()

        def wait_scatter(b):
            pltpu.make_async_copy(rows[b], acc.at[didx[b]], ssem[b]).wait()

        tbase = ebase + n_full * _K
        pltpu.async_copy(table_h.at[sidx.at[pl.ds(n_full * _K, tail_e)]],
                         rows_t, tsem[0])
        pltpu.async_copy(dst_h.at[pl.ds(tbase, tail_e)], didx_t, tsem[1])
        for c in range(_PF):
            start_fetch(c, c)
        plsc.subcore_barrier()
        n_main = (n_full - _PF) // _NBUF

        def body(i, carry):
            for b in range(_NBUF):
                c = _NBUF * i + b
                wait_fetch(c, b)
                pltpu.async_copy(rows[b], acc.at[didx[b]], ssem[b], add=True)
                bp = (b + _PF) % _NBUF
                if b + _PF < _NBUF:
                    @pl.when(i > 0)
                    def _():
                        wait_scatter(bp)
                else:
                    wait_scatter(bp)
                start_fetch(c + _PF, bp)
            return carry

        lax.fori_loop(0, n_main, body, 0)
        for c in range(_NBUF * n_main, n_full):
            b = c % _NBUF
            wait_fetch(c, b)
            pltpu.async_copy(rows[b], acc.at[didx[b]], ssem[b], add=True)
            if c + _PF < n_full:
                bp = (c + _PF) % _NBUF
                wait_scatter(bp)
                start_fetch(c + _PF, bp)
        pltpu.make_async_copy(table_h.at[sidx.at[pl.ds(n_full * _K, tail_e)]],
                              rows_t, tsem[0]).wait()
        pltpu.make_async_copy(dst_h.at[pl.ds(tbase, tail_e)], didx_t,
                              tsem[1]).wait()
        pltpu.async_copy(rows_t, acc.at[didx_t], tsem[2], add=True)
        for b in range(_NBUF):
            wait_scatter(b)
        pltpu.make_async_copy(rows_t, acc.at[didx_t], tsem[2]).wait()
        plsc.subcore_barrier()
        pltpu.sync_copy(acc.at[pl.ds(sid * rows_per_tile, rows_per_tile)],
                        out_h.at[cid, pl.ds(sid * rows_per_tile,
                                            rows_per_tile)])
        if tail_rows:
            @pl.when(sid == _NS - 1)
            def _():
                pltpu.sync_copy(acc.at[pl.ds(_NS * rows_per_tile, tail_rows)],
                                out_h.at[cid, pl.ds(_NS * rows_per_tile,
                                                    tail_rows)])

    return seg_kernel(table, src, dst)


def _gin_mlp(x, partials, w1, b1, g, bb, w2, b2, eps):
    n, d = x.shape

    def body(x_ref, p_ref, w1_ref, b1_ref, g_ref, bb_ref, w2_ref, b2_ref,
             eps_ref, out_ref):
        xin = (1.0 + eps_ref[0]) * x_ref[...] + p_ref[0] + p_ref[1]
        h = lax.dot_general(xin, w1_ref[...], (((1,), (1,)), ((), ())),
                            preferred_element_type=jnp.float32) + b1_ref[...]
        mu = jnp.mean(h, axis=0, keepdims=True)
        var = jnp.mean((h - mu) ** 2, axis=0, keepdims=True)
        hn = (h - mu) * lax.rsqrt(var + 1e-5) * g_ref[...] + bb_ref[...]
        hr = jnp.maximum(hn, 0.0)
        out_ref[...] = lax.dot_general(hr, w2_ref[...], (((1,), (1,)), ((), ())),
                                       preferred_element_type=jnp.float32
                                       ) + b2_ref[...]

    vspec = pl.BlockSpec(memory_space=pltpu.MemorySpace.VMEM)
    sspec = pl.BlockSpec(memory_space=pltpu.MemorySpace.SMEM)
    return pl.pallas_call(
        body,
        out_shape=jax.ShapeDtypeStruct((n, d), jnp.float32),
        in_specs=[vspec] * 8 + [sspec],
        out_specs=vspec,
    )(x, partials, w1, b1, g, bb, w2, b2, eps)


def _proj_gin_mlp(x, wp, bp, partials, w1, b1, g, bb, w2, b2, eps):
    n, d = x.shape

    def body(x_ref, wp_ref, bp_ref, p_ref, w1_ref, b1_ref, g_ref, bb_ref,
             w2_ref, b2_ref, eps_ref, out_ref):
        xp = lax.dot_general(x_ref[...], wp_ref[...], (((1,), (1,)), ((), ())),
                             preferred_element_type=jnp.float32) + bp_ref[...]
        xin = (1.0 + eps_ref[0]) * xp + p_ref[0] + p_ref[1]
        h = lax.dot_general(xin, w1_ref[...], (((1,), (1,)), ((), ())),
                            preferred_element_type=jnp.float32) + b1_ref[...]
        mu = jnp.mean(h, axis=0, keepdims=True)
        var = jnp.mean((h - mu) ** 2, axis=0, keepdims=True)
        hn = (h - mu) * lax.rsqrt(var + 1e-5) * g_ref[...] + bb_ref[...]
        hr = jnp.maximum(hn, 0.0)
        out_ref[...] = lax.dot_general(hr, w2_ref[...], (((1,), (1,)), ((), ())),
                                       preferred_element_type=jnp.float32
                                       ) + b2_ref[...]

    vspec = pl.BlockSpec(memory_space=pltpu.MemorySpace.VMEM)
    sspec = pl.BlockSpec(memory_space=pltpu.MemorySpace.SMEM)
    return pl.pallas_call(
        body,
        out_shape=jax.ShapeDtypeStruct((n, d), jnp.float32),
        in_specs=[vspec] * 10 + [sspec],
        out_specs=vspec,
    )(x, wp, bp, partials, w1, b1, g, bb, w2, b2, eps)


def kernel(feat_user, feat_item, edge_ui, edge_iu, W_proj, b_proj,
           ui_W1, ui_b1, ui_g, ui_bb, ui_W2, ui_b2,
           ret_W1, ret_b1, ret_g, ret_bb, ret_W2, ret_b2,
           eps_ui, eps_ret):
    n = feat_user.shape[0]
    p_item = _segment_sum_sc(feat_user, edge_ui[0], edge_ui[1], n)
    h_item = _gin_mlp(feat_item, p_item, ui_W1, ui_b1, ui_g, ui_bb,
                      ui_W2, ui_b2, eps_ui)
    p_user = _segment_sum_sc(h_item, edge_iu[0], edge_iu[1], n)
    h_user = _proj_gin_mlp(feat_user, W_proj, b_proj, p_user, ret_W1, ret_b1,
                           ret_g, ret_bb, ret_W2, ret_b2, eps_ret)
    return (h_user, h_item)
